# baseline (speedup 1.0000x reference)
import functools

import jax
import jax.numpy as jnp
from jax.experimental import pallas as pl
from jax.experimental.pallas import tpu as pltpu

_B = 2
_NGT = 4096
_NP1 = 2048
_NP2 = 4096
_NFPS = 2048
_NCLS = 8
_RB = 512


def _r16(x):
    return x.astype(jnp.bfloat16).astype(jnp.float32)


def _loss_body(gamma_ref, gt_ref, gtT_ref, gtR_ref, p2_ref, p1T_ref,
               loss_ref, cd2_ref, seg2_ref, glab_ref, fps_ref):
    f32 = jnp.float32
    gamma = gamma_ref[0, 0]

    iota_r = jax.lax.broadcasted_iota(jnp.int32, (32, 128), 0)
    iota_c = jax.lax.broadcasted_iota(jnp.int32, (32, 128), 1)
    iota2d = iota_r * 128 + iota_c
    for b in range(_B):
        fps_ref[b, 0:1, :] = gt_ref[b, 0:1, :]

    def fps_body(i, carry):
        d0, d1, p0c, p1c = carry
        out = []
        for b, dists, (xl, yl, zl) in ((0, d0, p0c), (1, d1, p1c)):
            gx = gtR_ref[b, 0]
            gy = gtR_ref[b, 1]
            gz = gtR_ref[b, 2]
            dx = gx - xl
            dy = gy - yl
            dz = gz - zl
            d = dx * dx + dy * dy + dz * dz
            dists = jnp.minimum(dists, d)
            m = jnp.max(dists)
            nxt = jnp.min(jnp.where(dists == m, iota2d, _NGT))
            row = gt_ref[b, pl.ds(nxt, 1), :]
            fps_ref[b, pl.ds(i, 1), :] = row
            out.append((dists, (row[:, 0:1], row[:, 1:2], row[:, 2:3])))
        (d0, p0c), (d1, p1c) = out
        return (d0, d1, p0c, p1c)

    init = (jnp.full((32, 128), 1e10, f32), jnp.full((32, 128), 1e10, f32),
            (gtT_ref[0, 0:1, 0:1], gtT_ref[0, 1:2, 0:1], gtT_ref[0, 2:3, 0:1]),
            (gtT_ref[1, 0:1, 0:1], gtT_ref[1, 1:2, 0:1], gtT_ref[1, 2:3, 0:1]))
    jax.lax.fori_loop(1, _NFPS, fps_body, init, unroll=False)

    lane_gt = jax.lax.broadcasted_iota(jnp.int32, (_RB, _NGT), 1)
    lane8 = jax.lax.broadcasted_iota(jnp.int32, (_RB, _NCLS), 1)
    sum_d1_2 = f32(0)
    sum_d2_2 = f32(0)
    seg2_sum = f32(0)
    for b in range(_B):
        gxc = gtT_ref[b, 0:1, :]
        gyc = gtT_ref[b, 1:2, :]
        gzc = gtT_ref[b, 2:3, :]
        glc = gtT_ref[b, 3:4, :]
        b2 = gxc * gxc + gyc * gyc + gzc * gzc
        gxc16 = _r16(gxc)
        gyc16 = _r16(gyc)
        gzc16 = _r16(gzc)
        key_gt = lane_gt * _NCLS + glc.astype(jnp.int32)
        colmin = jnp.full((1, _NGT), jnp.inf, f32)
        for rb in range(_NP2 // _RB):
            r0 = rb * _RB
            xr = p2_ref[b, r0:r0 + _RB, 0:1]
            yr = p2_ref[b, r0:r0 + _RB, 1:2]
            zr = p2_ref[b, r0:r0 + _RB, 2:3]
            a2 = xr * xr + yr * yr + zr * zr
            ab = _r16(xr) * gxc16 + _r16(yr) * gyc16 + _r16(zr) * gzc16
            d = jnp.maximum(a2 + b2 - 2.0 * ab, 0.0)
            rmin = jnp.min(d, axis=1, keepdims=True)
            kmin = jnp.min(jnp.where(d == rmin, key_gt, _NGT * _NCLS),
                           axis=1, keepdims=True)
            lab = (kmin & (_NCLS - 1)).astype(f32)
            glab_ref[b, r0:r0 + _RB, :] = lab
            sum_d1_2 = sum_d1_2 + jnp.sum(jnp.sqrt(rmin))
            colmin = jnp.minimum(colmin, jnp.min(d, axis=0, keepdims=True))
            logits = p2_ref[b, r0:r0 + _RB, 3:3 + _NCLS]
            mx = jnp.max(logits, axis=1, keepdims=True)
            sh = logits - mx
            logp = sh - jnp.log(jnp.sum(jnp.exp(sh), axis=1, keepdims=True))
            labi = kmin & (_NCLS - 1)
            logpt = jnp.sum(jnp.where(lane8 == labi, logp, 0.0), axis=1,
                            keepdims=True)
            pt = jnp.exp(logpt)
            seg2_sum = seg2_sum + jnp.sum(-((1.0 - pt) ** gamma) * logpt)
        sum_d2_2 = sum_d2_2 + jnp.sum(jnp.sqrt(colmin))
    cd2 = (sum_d1_2 / (_B * _NP2) + sum_d2_2 / (_B * _NGT)) / 2.0
    seg2 = seg2_sum / (_B * _NP2)

    row_iota = jax.lax.broadcasted_iota(jnp.int32, (_RB, _NP1), 0)
    sub8 = jax.lax.broadcasted_iota(jnp.int32, (_NCLS, _NP1), 0)
    sum_d1_1 = f32(0)
    sum_d2_1 = f32(0)
    seg1_sum = f32(0)
    for b in range(_B):
        pxc = p1T_ref[b, 0:1, :]
        pyc = p1T_ref[b, 1:2, :]
        pzc = p1T_ref[b, 2:3, :]
        c2 = pxc * pxc + pyc * pyc + pzc * pzc
        pxc16 = _r16(pxc)
        pyc16 = _r16(pyc)
        pzc16 = _r16(pzc)
        colmin = jnp.full((1, _NP1), jnp.inf, f32)
        colkey = jnp.zeros((1, _NP1), jnp.int32)
        for rb in range(_NFPS // _RB):
            r0 = rb * _RB
            xr = fps_ref[b, r0:r0 + _RB, 0:1]
            yr = fps_ref[b, r0:r0 + _RB, 1:2]
            zr = fps_ref[b, r0:r0 + _RB, 2:3]
            lr = fps_ref[b, r0:r0 + _RB, 3:4]
            a2 = xr * xr + yr * yr + zr * zr
            ab = _r16(xr) * pxc16 + _r16(yr) * pyc16 + _r16(zr) * pzc16
            d = jnp.maximum(a2 + c2 - 2.0 * ab, 0.0)
            rmin = jnp.min(d, axis=1, keepdims=True)
            sum_d2_1 = sum_d2_1 + jnp.sum(jnp.sqrt(rmin))
            bmin = jnp.min(d, axis=0, keepdims=True)
            keys = row_iota * _NCLS + lr.astype(jnp.int32)
            bkey = jnp.min(jnp.where(d == bmin, keys, _NFPS * _NCLS),
                           axis=0, keepdims=True)
            upd = bmin < colmin
            colkey = jnp.where(upd, bkey, colkey)
            colmin = jnp.where(upd, bmin, colmin)
        sum_d1_1 = sum_d1_1 + jnp.sum(jnp.sqrt(colmin))
        logits = p1T_ref[b, 3:3 + _NCLS, :]
        mx = jnp.max(logits, axis=0, keepdims=True)
        sh = logits - mx
        logp = sh - jnp.log(jnp.sum(jnp.exp(sh), axis=0, keepdims=True))
        labi = colkey & (_NCLS - 1)
        logpt = jnp.sum(jnp.where(sub8 == labi, logp, 0.0), axis=0,
                        keepdims=True)
        pt = jnp.exp(logpt)
        seg1_sum = seg1_sum + jnp.sum(-((1.0 - pt) ** gamma) * logpt)
    cd1 = (sum_d1_1 / (_B * _NP1) + sum_d2_1 / (_B * _NFPS)) / 2.0
    seg1 = seg1_sum / (_B * _NP1)

    loss_ref[:, :] = ((cd1 + cd2) * 1000.0 + (seg1 + seg2) * 100.0).reshape(1, 1)
    cd2_ref[:, :] = cd2.reshape(1, 1)
    seg2_ref[:, :] = seg2.reshape(1, 1)


@functools.partial(jax.jit, static_argnames=("interpret",))
def _run(gamma, gt, gtT, gtR, p2, p1T, interpret=False):
    f32 = jnp.float32
    out_shapes = (
        jax.ShapeDtypeStruct((1, 1), f32),
        jax.ShapeDtypeStruct((1, 1), f32),
        jax.ShapeDtypeStruct((1, 1), f32),
        jax.ShapeDtypeStruct((_B, _NP2, 1), f32),
    )
    return pl.pallas_call(
        _loss_body,
        out_shape=out_shapes,
        in_specs=[
            pl.BlockSpec(memory_space=pltpu.SMEM),
            pl.BlockSpec(memory_space=pltpu.VMEM),
            pl.BlockSpec(memory_space=pltpu.VMEM),
            pl.BlockSpec(memory_space=pltpu.VMEM),
            pl.BlockSpec(memory_space=pltpu.VMEM),
            pl.BlockSpec(memory_space=pltpu.VMEM),
        ],
        scratch_shapes=[pltpu.VMEM((_B, _NFPS, 4), f32)],
        interpret=interpret,
    )(gamma, gt, gtT, gtR, p2, p1T)


def kernel(p0, p1, p2, gt, epoch, interpret=False):
    del p0
    gamma = jnp.clip(5.0 * (epoch / 200.0), 0.0, 20.0)
    gamma = jnp.asarray(gamma, jnp.float32).reshape(1, 1)
    gtT = jnp.transpose(gt, (0, 2, 1))
    gtR = gtT.reshape(_B, 4, 32, 128)
    p1T = jnp.transpose(p1, (0, 2, 1))
    loss_all, cd2, seg2, glab = _run(gamma, gt, gtT, gtR, p2, p1T,
                                     interpret=interpret)
    pred_label = p2[:, :, 3:]
    return (loss_all.reshape(()), cd2.reshape(()), seg2.reshape(()),
            pred_label, glab.reshape(_B, _NP2))


# speedup vs baseline: 2.9429x; 2.9429x over previous
import functools

import jax
import jax.numpy as jnp
from jax.experimental import pallas as pl
from jax.experimental.pallas import tpu as pltpu

_B = 2
_NGT = 4096
_NP1 = 2048
_NP2 = 4096
_NFPS = 2048
_NCLS = 8
_RB = 512


def _r16(x):
    return x.astype(jnp.bfloat16).astype(jnp.float32)


def _loss_body(gamma_ref, gt_ref, gtT_ref, gtR_ref, p2_ref, p1T_ref,
               loss_ref, cd2_ref, seg2_ref, glab_ref, fps_ref):
    f32 = jnp.float32
    gamma = gamma_ref[0, 0]

    neg_inf = f32(-jnp.inf)
    for b in range(_B):
        fps_ref[b, 0:1, :] = gt_ref[b, 0:1, :]

    def fps_body(i, carry):
        d0, d1, p0c, p1c = carry
        out = []
        for b, dists, (xl, yl, zl) in ((0, d0, p0c), (1, d1, p1c)):
            gx = gtR_ref[b, 0]
            gy = gtR_ref[b, 1]
            gz = gtR_ref[b, 2]
            gl = gtR_ref[b, 3]
            dx = gx - xl
            dy = gy - yl
            dz = gz - zl
            d = dx * dx + dy * dy + dz * dz
            dists = jnp.minimum(dists, d)
            colmax = jnp.max(dists, axis=0, keepdims=True)
            rowsel = dists == colmax
            cwx = jnp.max(jnp.where(rowsel, gx, neg_inf), axis=0,
                          keepdims=True)
            cwy = jnp.max(jnp.where(rowsel, gy, neg_inf), axis=0,
                          keepdims=True)
            cwz = jnp.max(jnp.where(rowsel, gz, neg_inf), axis=0,
                          keepdims=True)
            cwl = jnp.max(jnp.where(rowsel, gl, neg_inf), axis=0,
                          keepdims=True)
            m = jnp.max(colmax, axis=1, keepdims=True)
            selc = colmax == m
            stack = jnp.concatenate([cwx, cwy, cwz, cwl], axis=0)
            win = jnp.max(jnp.where(selc, stack, neg_inf), axis=1,
                          keepdims=True)
            fps_ref[b, pl.ds(i, 1), 0:1] = win[0:1, :]
            fps_ref[b, pl.ds(i, 1), 1:2] = win[1:2, :]
            fps_ref[b, pl.ds(i, 1), 2:3] = win[2:3, :]
            fps_ref[b, pl.ds(i, 1), 3:4] = win[3:4, :]
            out.append((dists, (win[0:1, :], win[1:2, :], win[2:3, :])))
        (d0, p0c), (d1, p1c) = out
        return (d0, d1, p0c, p1c)

    init = (jnp.full((32, 128), 1e10, f32), jnp.full((32, 128), 1e10, f32),
            (gtT_ref[0, 0:1, 0:1], gtT_ref[0, 1:2, 0:1], gtT_ref[0, 2:3, 0:1]),
            (gtT_ref[1, 0:1, 0:1], gtT_ref[1, 1:2, 0:1], gtT_ref[1, 2:3, 0:1]))
    jax.lax.fori_loop(1, _NFPS, fps_body, init, unroll=False)

    lane_gt = jax.lax.broadcasted_iota(jnp.int32, (_RB, _NGT), 1)
    lane8 = jax.lax.broadcasted_iota(jnp.int32, (_RB, _NCLS), 1)
    sum_d1_2 = f32(0)
    sum_d2_2 = f32(0)
    seg2_sum = f32(0)
    for b in range(_B):
        gxc = gtT_ref[b, 0:1, :]
        gyc = gtT_ref[b, 1:2, :]
        gzc = gtT_ref[b, 2:3, :]
        glc = gtT_ref[b, 3:4, :]
        b2 = gxc * gxc + gyc * gyc + gzc * gzc
        gxc16 = _r16(gxc)
        gyc16 = _r16(gyc)
        gzc16 = _r16(gzc)
        key_gt = lane_gt * _NCLS + glc.astype(jnp.int32)
        colmin = jnp.full((1, _NGT), jnp.inf, f32)
        for rb in range(_NP2 // _RB):
            r0 = rb * _RB
            xr = p2_ref[b, r0:r0 + _RB, 0:1]
            yr = p2_ref[b, r0:r0 + _RB, 1:2]
            zr = p2_ref[b, r0:r0 + _RB, 2:3]
            a2 = xr * xr + yr * yr + zr * zr
            ab = _r16(xr) * gxc16 + _r16(yr) * gyc16 + _r16(zr) * gzc16
            d = jnp.maximum(a2 + b2 - 2.0 * ab, 0.0)
            rmin = jnp.min(d, axis=1, keepdims=True)
            kmin = jnp.min(jnp.where(d == rmin, key_gt, _NGT * _NCLS),
                           axis=1, keepdims=True)
            lab = (kmin & (_NCLS - 1)).astype(f32)
            glab_ref[b, r0:r0 + _RB, :] = lab
            sum_d1_2 = sum_d1_2 + jnp.sum(jnp.sqrt(rmin))
            colmin = jnp.minimum(colmin, jnp.min(d, axis=0, keepdims=True))
            logits = p2_ref[b, r0:r0 + _RB, 3:3 + _NCLS]
            mx = jnp.max(logits, axis=1, keepdims=True)
            sh = logits - mx
            logp = sh - jnp.log(jnp.sum(jnp.exp(sh), axis=1, keepdims=True))
            labi = kmin & (_NCLS - 1)
            logpt = jnp.sum(jnp.where(lane8 == labi, logp, 0.0), axis=1,
                            keepdims=True)
            pt = jnp.exp(logpt)
            seg2_sum = seg2_sum + jnp.sum(-((1.0 - pt) ** gamma) * logpt)
        sum_d2_2 = sum_d2_2 + jnp.sum(jnp.sqrt(colmin))
    cd2 = (sum_d1_2 / (_B * _NP2) + sum_d2_2 / (_B * _NGT)) / 2.0
    seg2 = seg2_sum / (_B * _NP2)

    row_iota = jax.lax.broadcasted_iota(jnp.int32, (_RB, _NP1), 0)
    sub8 = jax.lax.broadcasted_iota(jnp.int32, (_NCLS, _NP1), 0)
    sum_d1_1 = f32(0)
    sum_d2_1 = f32(0)
    seg1_sum = f32(0)
    for b in range(_B):
        pxc = p1T_ref[b, 0:1, :]
        pyc = p1T_ref[b, 1:2, :]
        pzc = p1T_ref[b, 2:3, :]
        c2 = pxc * pxc + pyc * pyc + pzc * pzc
        pxc16 = _r16(pxc)
        pyc16 = _r16(pyc)
        pzc16 = _r16(pzc)
        colmin = jnp.full((1, _NP1), jnp.inf, f32)
        colkey = jnp.zeros((1, _NP1), jnp.int32)
        for rb in range(_NFPS // _RB):
            r0 = rb * _RB
            xr = fps_ref[b, r0:r0 + _RB, 0:1]
            yr = fps_ref[b, r0:r0 + _RB, 1:2]
            zr = fps_ref[b, r0:r0 + _RB, 2:3]
            lr = fps_ref[b, r0:r0 + _RB, 3:4]
            a2 = xr * xr + yr * yr + zr * zr
            ab = _r16(xr) * pxc16 + _r16(yr) * pyc16 + _r16(zr) * pzc16
            d = jnp.maximum(a2 + c2 - 2.0 * ab, 0.0)
            rmin = jnp.min(d, axis=1, keepdims=True)
            sum_d2_1 = sum_d2_1 + jnp.sum(jnp.sqrt(rmin))
            bmin = jnp.min(d, axis=0, keepdims=True)
            keys = row_iota * _NCLS + lr.astype(jnp.int32)
            bkey = jnp.min(jnp.where(d == bmin, keys, _NFPS * _NCLS),
                           axis=0, keepdims=True)
            upd = bmin < colmin
            colkey = jnp.where(upd, bkey, colkey)
            colmin = jnp.where(upd, bmin, colmin)
        sum_d1_1 = sum_d1_1 + jnp.sum(jnp.sqrt(colmin))
        logits = p1T_ref[b, 3:3 + _NCLS, :]
        mx = jnp.max(logits, axis=0, keepdims=True)
        sh = logits - mx
        logp = sh - jnp.log(jnp.sum(jnp.exp(sh), axis=0, keepdims=True))
        labi = colkey & (_NCLS - 1)
        logpt = jnp.sum(jnp.where(sub8 == labi, logp, 0.0), axis=0,
                        keepdims=True)
        pt = jnp.exp(logpt)
        seg1_sum = seg1_sum + jnp.sum(-((1.0 - pt) ** gamma) * logpt)
    cd1 = (sum_d1_1 / (_B * _NP1) + sum_d2_1 / (_B * _NFPS)) / 2.0
    seg1 = seg1_sum / (_B * _NP1)

    loss_ref[:, :] = ((cd1 + cd2) * 1000.0 + (seg1 + seg2) * 100.0).reshape(1, 1)
    cd2_ref[:, :] = cd2.reshape(1, 1)
    seg2_ref[:, :] = seg2.reshape(1, 1)


@functools.partial(jax.jit, static_argnames=("interpret",))
def _run(gamma, gt, gtT, gtR, p2, p1T, interpret=False):
    f32 = jnp.float32
    out_shapes = (
        jax.ShapeDtypeStruct((1, 1), f32),
        jax.ShapeDtypeStruct((1, 1), f32),
        jax.ShapeDtypeStruct((1, 1), f32),
        jax.ShapeDtypeStruct((_B, _NP2, 1), f32),
    )
    return pl.pallas_call(
        _loss_body,
        out_shape=out_shapes,
        in_specs=[
            pl.BlockSpec(memory_space=pltpu.SMEM),
            pl.BlockSpec(memory_space=pltpu.VMEM),
            pl.BlockSpec(memory_space=pltpu.VMEM),
            pl.BlockSpec(memory_space=pltpu.VMEM),
            pl.BlockSpec(memory_space=pltpu.VMEM),
            pl.BlockSpec(memory_space=pltpu.VMEM),
        ],
        scratch_shapes=[pltpu.VMEM((_B, _NFPS, 4), f32)],
        interpret=interpret,
    )(gamma, gt, gtT, gtR, p2, p1T)


def kernel(p0, p1, p2, gt, epoch, interpret=False):
    del p0
    gamma = jnp.clip(5.0 * (epoch / 200.0), 0.0, 20.0)
    gamma = jnp.asarray(gamma, jnp.float32).reshape(1, 1)
    gtT = jnp.transpose(gt, (0, 2, 1))
    gtR = gtT.reshape(_B, 4, 32, 128)
    p1T = jnp.transpose(p1, (0, 2, 1))
    loss_all, cd2, seg2, glab = _run(gamma, gt, gtT, gtR, p2, p1T,
                                     interpret=interpret)
    pred_label = p2[:, :, 3:]
    return (loss_all.reshape(()), cd2.reshape(()), seg2.reshape(()),
            pred_label, glab.reshape(_B, _NP2))


# speedup vs baseline: 2.9477x; 1.0016x over previous
import functools

import jax
import jax.numpy as jnp
from jax.experimental import pallas as pl
from jax.experimental.pallas import tpu as pltpu

_B = 2
_NGT = 4096
_NP1 = 2048
_NP2 = 4096
_NFPS = 2048
_NCLS = 8
_RB = 512


def _r16(x):
    return x.astype(jnp.bfloat16).astype(jnp.float32)


def _loss_body(gamma_ref, gt_ref, gtT_ref, gtR_ref, p2_ref, p1T_ref,
               loss_ref, cd2_ref, seg2_ref, glab_ref, fps_ref):
    f32 = jnp.float32
    gamma = gamma_ref[0, 0]

    neg_inf = f32(-jnp.inf)
    for b in range(_B):
        fps_ref[b, 0:1, :] = gt_ref[b, 0:1, :]

    def fps_body(i, carry):
        dists, xl, yl, zl = carry
        gx = gtR_ref[:, 0]
        gy = gtR_ref[:, 1]
        gz = gtR_ref[:, 2]
        gl = gtR_ref[:, 3]
        dx = gx - xl
        dy = gy - yl
        dz = gz - zl
        d = dx * dx + dy * dy + dz * dz
        dists = jnp.minimum(dists, d)
        colmax = jnp.max(dists, axis=1, keepdims=True)
        rowsel = dists == colmax
        cwx = jnp.max(jnp.where(rowsel, gx, neg_inf), axis=1, keepdims=True)
        cwy = jnp.max(jnp.where(rowsel, gy, neg_inf), axis=1, keepdims=True)
        cwz = jnp.max(jnp.where(rowsel, gz, neg_inf), axis=1, keepdims=True)
        cwl = jnp.max(jnp.where(rowsel, gl, neg_inf), axis=1, keepdims=True)
        m = jnp.max(colmax, axis=2, keepdims=True)
        selc = colmax == m
        stack = jnp.concatenate([cwx, cwy, cwz, cwl], axis=1)
        win = jnp.max(jnp.where(selc, stack, neg_inf), axis=2,
                      keepdims=True)
        for b in range(_B):
            fps_ref[b, pl.ds(i, 1), 0:1] = win[b, 0:1, :]
            fps_ref[b, pl.ds(i, 1), 1:2] = win[b, 1:2, :]
            fps_ref[b, pl.ds(i, 1), 2:3] = win[b, 2:3, :]
            fps_ref[b, pl.ds(i, 1), 3:4] = win[b, 3:4, :]
        return (dists, win[:, 0:1, :], win[:, 1:2, :], win[:, 2:3, :])

    init = (jnp.full((_B, 32, 128), 1e10, f32),
            gtT_ref[:, 0:1, 0:1], gtT_ref[:, 1:2, 0:1], gtT_ref[:, 2:3, 0:1])
    jax.lax.fori_loop(1, _NFPS, fps_body, init, unroll=False)

    lane_gt = jax.lax.broadcasted_iota(jnp.int32, (_RB, _NGT), 1)
    lane8 = jax.lax.broadcasted_iota(jnp.int32, (_RB, _NCLS), 1)
    sum_d1_2 = f32(0)
    sum_d2_2 = f32(0)
    seg2_sum = f32(0)
    for b in range(_B):
        gxc = gtT_ref[b, 0:1, :]
        gyc = gtT_ref[b, 1:2, :]
        gzc = gtT_ref[b, 2:3, :]
        glc = gtT_ref[b, 3:4, :]
        b2 = gxc * gxc + gyc * gyc + gzc * gzc
        gxc16 = _r16(gxc)
        gyc16 = _r16(gyc)
        gzc16 = _r16(gzc)
        key_gt = lane_gt * _NCLS + glc.astype(jnp.int32)
        colmin = jnp.full((1, _NGT), jnp.inf, f32)
        for rb in range(_NP2 // _RB):
            r0 = rb * _RB
            xr = p2_ref[b, r0:r0 + _RB, 0:1]
            yr = p2_ref[b, r0:r0 + _RB, 1:2]
            zr = p2_ref[b, r0:r0 + _RB, 2:3]
            a2 = xr * xr + yr * yr + zr * zr
            ab = _r16(xr) * gxc16 + _r16(yr) * gyc16 + _r16(zr) * gzc16
            d = jnp.maximum(a2 + b2 - 2.0 * ab, 0.0)
            rmin = jnp.min(d, axis=1, keepdims=True)
            kmin = jnp.min(jnp.where(d == rmin, key_gt, _NGT * _NCLS),
                           axis=1, keepdims=True)
            lab = (kmin & (_NCLS - 1)).astype(f32)
            glab_ref[b, r0:r0 + _RB, :] = lab
            sum_d1_2 = sum_d1_2 + jnp.sum(jnp.sqrt(rmin))
            colmin = jnp.minimum(colmin, jnp.min(d, axis=0, keepdims=True))
            logits = p2_ref[b, r0:r0 + _RB, 3:3 + _NCLS]
            mx = jnp.max(logits, axis=1, keepdims=True)
            sh = logits - mx
            logp = sh - jnp.log(jnp.sum(jnp.exp(sh), axis=1, keepdims=True))
            labi = kmin & (_NCLS - 1)
            logpt = jnp.sum(jnp.where(lane8 == labi, logp, 0.0), axis=1,
                            keepdims=True)
            pt = jnp.exp(logpt)
            seg2_sum = seg2_sum + jnp.sum(-((1.0 - pt) ** gamma) * logpt)
        sum_d2_2 = sum_d2_2 + jnp.sum(jnp.sqrt(colmin))
    cd2 = (sum_d1_2 / (_B * _NP2) + sum_d2_2 / (_B * _NGT)) / 2.0
    seg2 = seg2_sum / (_B * _NP2)

    row_iota = jax.lax.broadcasted_iota(jnp.int32, (_RB, _NP1), 0)
    sub8 = jax.lax.broadcasted_iota(jnp.int32, (_NCLS, _NP1), 0)
    sum_d1_1 = f32(0)
    sum_d2_1 = f32(0)
    seg1_sum = f32(0)
    for b in range(_B):
        pxc = p1T_ref[b, 0:1, :]
        pyc = p1T_ref[b, 1:2, :]
        pzc = p1T_ref[b, 2:3, :]
        c2 = pxc * pxc + pyc * pyc + pzc * pzc
        pxc16 = _r16(pxc)
        pyc16 = _r16(pyc)
        pzc16 = _r16(pzc)
        colmin = jnp.full((1, _NP1), jnp.inf, f32)
        colkey = jnp.zeros((1, _NP1), jnp.int32)
        for rb in range(_NFPS // _RB):
            r0 = rb * _RB
            xr = fps_ref[b, r0:r0 + _RB, 0:1]
            yr = fps_ref[b, r0:r0 + _RB, 1:2]
            zr = fps_ref[b, r0:r0 + _RB, 2:3]
            lr = fps_ref[b, r0:r0 + _RB, 3:4]
            a2 = xr * xr + yr * yr + zr * zr
            ab = _r16(xr) * pxc16 + _r16(yr) * pyc16 + _r16(zr) * pzc16
            d = jnp.maximum(a2 + c2 - 2.0 * ab, 0.0)
            rmin = jnp.min(d, axis=1, keepdims=True)
            sum_d2_1 = sum_d2_1 + jnp.sum(jnp.sqrt(rmin))
            bmin = jnp.min(d, axis=0, keepdims=True)
            keys = row_iota * _NCLS + lr.astype(jnp.int32)
            bkey = jnp.min(jnp.where(d == bmin, keys, _NFPS * _NCLS),
                           axis=0, keepdims=True)
            upd = bmin < colmin
            colkey = jnp.where(upd, bkey, colkey)
            colmin = jnp.where(upd, bmin, colmin)
        sum_d1_1 = sum_d1_1 + jnp.sum(jnp.sqrt(colmin))
        logits = p1T_ref[b, 3:3 + _NCLS, :]
        mx = jnp.max(logits, axis=0, keepdims=True)
        sh = logits - mx
        logp = sh - jnp.log(jnp.sum(jnp.exp(sh), axis=0, keepdims=True))
        labi = colkey & (_NCLS - 1)
        logpt = jnp.sum(jnp.where(sub8 == labi, logp, 0.0), axis=0,
                        keepdims=True)
        pt = jnp.exp(logpt)
        seg1_sum = seg1_sum + jnp.sum(-((1.0 - pt) ** gamma) * logpt)
    cd1 = (sum_d1_1 / (_B * _NP1) + sum_d2_1 / (_B * _NFPS)) / 2.0
    seg1 = seg1_sum / (_B * _NP1)

    loss_ref[:, :] = ((cd1 + cd2) * 1000.0 + (seg1 + seg2) * 100.0).reshape(1, 1)
    cd2_ref[:, :] = cd2.reshape(1, 1)
    seg2_ref[:, :] = seg2.reshape(1, 1)


@functools.partial(jax.jit, static_argnames=("interpret",))
def _run(gamma, gt, gtT, gtR, p2, p1T, interpret=False):
    f32 = jnp.float32
    out_shapes = (
        jax.ShapeDtypeStruct((1, 1), f32),
        jax.ShapeDtypeStruct((1, 1), f32),
        jax.ShapeDtypeStruct((1, 1), f32),
        jax.ShapeDtypeStruct((_B, _NP2, 1), f32),
    )
    return pl.pallas_call(
        _loss_body,
        out_shape=out_shapes,
        in_specs=[
            pl.BlockSpec(memory_space=pltpu.SMEM),
            pl.BlockSpec(memory_space=pltpu.VMEM),
            pl.BlockSpec(memory_space=pltpu.VMEM),
            pl.BlockSpec(memory_space=pltpu.VMEM),
            pl.BlockSpec(memory_space=pltpu.VMEM),
            pl.BlockSpec(memory_space=pltpu.VMEM),
        ],
        scratch_shapes=[pltpu.VMEM((_B, _NFPS, 4), f32)],
        interpret=interpret,
    )(gamma, gt, gtT, gtR, p2, p1T)


def kernel(p0, p1, p2, gt, epoch, interpret=False):
    del p0
    gamma = jnp.clip(5.0 * (epoch / 200.0), 0.0, 20.0)
    gamma = jnp.asarray(gamma, jnp.float32).reshape(1, 1)
    gtT = jnp.transpose(gt, (0, 2, 1))
    gtR = gtT.reshape(_B, 4, 32, 128)
    p1T = jnp.transpose(p1, (0, 2, 1))
    loss_all, cd2, seg2, glab = _run(gamma, gt, gtT, gtR, p2, p1T,
                                     interpret=interpret)
    pred_label = p2[:, :, 3:]
    return (loss_all.reshape(()), cd2.reshape(()), seg2.reshape(()),
            pred_label, glab.reshape(_B, _NP2))


# speedup vs baseline: 3.3252x; 1.1281x over previous
import functools

import jax
import jax.numpy as jnp
from jax.experimental import pallas as pl
from jax.experimental.pallas import tpu as pltpu

_B = 2
_NGT = 4096
_NP1 = 2048
_NP2 = 4096
_NFPS = 2048
_NCLS = 8
_RB = 512


def _r16(x):
    return x.astype(jnp.bfloat16).astype(jnp.float32)


def _loss_body(gamma_ref, gt_ref, gtT_ref, gtR_ref, p2_ref, p1T_ref,
               loss_ref, cd2_ref, seg2_ref, glab_ref, fps_ref):
    f32 = jnp.float32
    gamma = gamma_ref[0, 0]

    neg_inf = f32(-jnp.inf)
    for b in range(_B):
        fps_ref[b, 0:1, :] = gt_ref[b, 0:1, :]

    def fps_body(i, carry):
        dists, xl, yl, zl = carry
        gx = gtR_ref[:, 0]
        gy = gtR_ref[:, 1]
        gz = gtR_ref[:, 2]
        gl = gtR_ref[:, 3]
        dx = gx - xl
        dy = gy - yl
        dz = gz - zl
        d = dx * dx + dy * dy + dz * dz
        dists = jnp.minimum(dists, d)
        colmax = jnp.max(dists, axis=1, keepdims=True)
        rowsel = dists == colmax
        cwx = jnp.max(jnp.where(rowsel, gx, neg_inf), axis=1, keepdims=True)
        cwy = jnp.max(jnp.where(rowsel, gy, neg_inf), axis=1, keepdims=True)
        cwz = jnp.max(jnp.where(rowsel, gz, neg_inf), axis=1, keepdims=True)
        cwl = jnp.max(jnp.where(rowsel, gl, neg_inf), axis=1, keepdims=True)
        m = jnp.max(colmax, axis=2, keepdims=True)
        selc = colmax == m
        stack = jnp.concatenate([cwx, cwy, cwz, cwl], axis=1)
        win = jnp.max(jnp.where(selc, stack, neg_inf), axis=2,
                      keepdims=True)
        for b in range(_B):
            fps_ref[b, pl.ds(i, 1), 0:1] = win[b, 0:1, :]
            fps_ref[b, pl.ds(i, 1), 1:2] = win[b, 1:2, :]
            fps_ref[b, pl.ds(i, 1), 2:3] = win[b, 2:3, :]
            fps_ref[b, pl.ds(i, 1), 3:4] = win[b, 3:4, :]
        return (dists, win[:, 0:1, :], win[:, 1:2, :], win[:, 2:3, :])

    init = (jnp.full((_B, 32, 128), 1e10, f32),
            gtT_ref[:, 0:1, 0:1], gtT_ref[:, 1:2, 0:1], gtT_ref[:, 2:3, 0:1])
    jax.lax.fori_loop(1, _NFPS, fps_body, init, unroll=4)

    lane_gt = jax.lax.broadcasted_iota(jnp.int32, (_RB, _NGT), 1)
    lane8 = jax.lax.broadcasted_iota(jnp.int32, (_RB, _NCLS), 1)
    sum_d1_2 = f32(0)
    sum_d2_2 = f32(0)
    seg2_sum = f32(0)
    for b in range(_B):
        gxc = gtT_ref[b, 0:1, :]
        gyc = gtT_ref[b, 1:2, :]
        gzc = gtT_ref[b, 2:3, :]
        glc = gtT_ref[b, 3:4, :]
        b2 = gxc * gxc + gyc * gyc + gzc * gzc
        gxc16 = _r16(gxc)
        gyc16 = _r16(gyc)
        gzc16 = _r16(gzc)
        key_gt = lane_gt * _NCLS + glc.astype(jnp.int32)
        colmin = jnp.full((1, _NGT), jnp.inf, f32)
        for rb in range(_NP2 // _RB):
            r0 = rb * _RB
            xr = p2_ref[b, r0:r0 + _RB, 0:1]
            yr = p2_ref[b, r0:r0 + _RB, 1:2]
            zr = p2_ref[b, r0:r0 + _RB, 2:3]
            a2 = xr * xr + yr * yr + zr * zr
            ab = _r16(xr) * gxc16 + _r16(yr) * gyc16 + _r16(zr) * gzc16
            d = jnp.maximum(a2 + b2 - 2.0 * ab, 0.0)
            rmin = jnp.min(d, axis=1, keepdims=True)
            kmin = jnp.min(jnp.where(d == rmin, key_gt, _NGT * _NCLS),
                           axis=1, keepdims=True)
            lab = (kmin & (_NCLS - 1)).astype(f32)
            glab_ref[b, r0:r0 + _RB, :] = lab
            sum_d1_2 = sum_d1_2 + jnp.sum(jnp.sqrt(rmin))
            colmin = jnp.minimum(colmin, jnp.min(d, axis=0, keepdims=True))
            logits = p2_ref[b, r0:r0 + _RB, 3:3 + _NCLS]
            mx = jnp.max(logits, axis=1, keepdims=True)
            sh = logits - mx
            logp = sh - jnp.log(jnp.sum(jnp.exp(sh), axis=1, keepdims=True))
            labi = kmin & (_NCLS - 1)
            logpt = jnp.sum(jnp.where(lane8 == labi, logp, 0.0), axis=1,
                            keepdims=True)
            pt = jnp.exp(logpt)
            seg2_sum = seg2_sum + jnp.sum(-((1.0 - pt) ** gamma) * logpt)
        sum_d2_2 = sum_d2_2 + jnp.sum(jnp.sqrt(colmin))
    cd2 = (sum_d1_2 / (_B * _NP2) + sum_d2_2 / (_B * _NGT)) / 2.0
    seg2 = seg2_sum / (_B * _NP2)

    row_iota = jax.lax.broadcasted_iota(jnp.int32, (_RB, _NP1), 0)
    sub8 = jax.lax.broadcasted_iota(jnp.int32, (_NCLS, _NP1), 0)
    sum_d1_1 = f32(0)
    sum_d2_1 = f32(0)
    seg1_sum = f32(0)
    for b in range(_B):
        pxc = p1T_ref[b, 0:1, :]
        pyc = p1T_ref[b, 1:2, :]
        pzc = p1T_ref[b, 2:3, :]
        c2 = pxc * pxc + pyc * pyc + pzc * pzc
        pxc16 = _r16(pxc)
        pyc16 = _r16(pyc)
        pzc16 = _r16(pzc)
        colmin = jnp.full((1, _NP1), jnp.inf, f32)
        colkey = jnp.zeros((1, _NP1), jnp.int32)
        for rb in range(_NFPS // _RB):
            r0 = rb * _RB
            xr = fps_ref[b, r0:r0 + _RB, 0:1]
            yr = fps_ref[b, r0:r0 + _RB, 1:2]
            zr = fps_ref[b, r0:r0 + _RB, 2:3]
            lr = fps_ref[b, r0:r0 + _RB, 3:4]
            a2 = xr * xr + yr * yr + zr * zr
            ab = _r16(xr) * pxc16 + _r16(yr) * pyc16 + _r16(zr) * pzc16
            d = jnp.maximum(a2 + c2 - 2.0 * ab, 0.0)
            rmin = jnp.min(d, axis=1, keepdims=True)
            sum_d2_1 = sum_d2_1 + jnp.sum(jnp.sqrt(rmin))
            bmin = jnp.min(d, axis=0, keepdims=True)
            keys = row_iota * _NCLS + lr.astype(jnp.int32)
            bkey = jnp.min(jnp.where(d == bmin, keys, _NFPS * _NCLS),
                           axis=0, keepdims=True)
            upd = bmin < colmin
            colkey = jnp.where(upd, bkey, colkey)
            colmin = jnp.where(upd, bmin, colmin)
        sum_d1_1 = sum_d1_1 + jnp.sum(jnp.sqrt(colmin))
        logits = p1T_ref[b, 3:3 + _NCLS, :]
        mx = jnp.max(logits, axis=0, keepdims=True)
        sh = logits - mx
        logp = sh - jnp.log(jnp.sum(jnp.exp(sh), axis=0, keepdims=True))
        labi = colkey & (_NCLS - 1)
        logpt = jnp.sum(jnp.where(sub8 == labi, logp, 0.0), axis=0,
                        keepdims=True)
        pt = jnp.exp(logpt)
        seg1_sum = seg1_sum + jnp.sum(-((1.0 - pt) ** gamma) * logpt)
    cd1 = (sum_d1_1 / (_B * _NP1) + sum_d2_1 / (_B * _NFPS)) / 2.0
    seg1 = seg1_sum / (_B * _NP1)

    loss_ref[:, :] = ((cd1 + cd2) * 1000.0 + (seg1 + seg2) * 100.0).reshape(1, 1)
    cd2_ref[:, :] = cd2.reshape(1, 1)
    seg2_ref[:, :] = seg2.reshape(1, 1)


@functools.partial(jax.jit, static_argnames=("interpret",))
def _run(gamma, gt, gtT, gtR, p2, p1T, interpret=False):
    f32 = jnp.float32
    out_shapes = (
        jax.ShapeDtypeStruct((1, 1), f32),
        jax.ShapeDtypeStruct((1, 1), f32),
        jax.ShapeDtypeStruct((1, 1), f32),
        jax.ShapeDtypeStruct((_B, _NP2, 1), f32),
    )
    return pl.pallas_call(
        _loss_body,
        out_shape=out_shapes,
        in_specs=[
            pl.BlockSpec(memory_space=pltpu.SMEM),
            pl.BlockSpec(memory_space=pltpu.VMEM),
            pl.BlockSpec(memory_space=pltpu.VMEM),
            pl.BlockSpec(memory_space=pltpu.VMEM),
            pl.BlockSpec(memory_space=pltpu.VMEM),
            pl.BlockSpec(memory_space=pltpu.VMEM),
        ],
        scratch_shapes=[pltpu.VMEM((_B, _NFPS, 4), f32)],
        interpret=interpret,
    )(gamma, gt, gtT, gtR, p2, p1T)


def kernel(p0, p1, p2, gt, epoch, interpret=False):
    del p0
    gamma = jnp.clip(5.0 * (epoch / 200.0), 0.0, 20.0)
    gamma = jnp.asarray(gamma, jnp.float32).reshape(1, 1)
    gtT = jnp.transpose(gt, (0, 2, 1))
    gtR = gtT.reshape(_B, 4, 32, 128)
    p1T = jnp.transpose(p1, (0, 2, 1))
    loss_all, cd2, seg2, glab = _run(gamma, gt, gtT, gtR, p2, p1T,
                                     interpret=interpret)
    pred_label = p2[:, :, 3:]
    return (loss_all.reshape(()), cd2.reshape(()), seg2.reshape(()),
            pred_label, glab.reshape(_B, _NP2))


# speedup vs baseline: 3.3975x; 1.0218x over previous
import functools

import jax
import jax.numpy as jnp
from jax.experimental import pallas as pl
from jax.experimental.pallas import tpu as pltpu

_B = 2
_NGT = 4096
_NP1 = 2048
_NP2 = 4096
_NFPS = 2048
_NCLS = 8
_RB = 512


def _r16(x):
    return x.astype(jnp.bfloat16).astype(jnp.float32)


def _loss_body(gamma_ref, gt_ref, gtT_ref, gtR_ref, p2_ref, p1T_ref,
               loss_ref, cd2_ref, seg2_ref, glab_ref, fps_ref):
    f32 = jnp.float32
    gamma = gamma_ref[0, 0]

    neg_inf = f32(-jnp.inf)
    for b in range(_B):
        fps_ref[b, 0:1, :] = gt_ref[b, 0:1, :]

    def fps_body(i, carry):
        dists, xl, yl, zl = carry
        gx = gtR_ref[:, 0]
        gy = gtR_ref[:, 1]
        gz = gtR_ref[:, 2]
        gl = gtR_ref[:, 3]
        dx = gx - xl
        dy = gy - yl
        dz = gz - zl
        d = dx * dx + dy * dy + dz * dz
        dists = jnp.minimum(dists, d)
        colmax = jnp.max(dists, axis=1, keepdims=True)
        rowsel = dists == colmax
        cwx = jnp.max(jnp.where(rowsel, gx, neg_inf), axis=1, keepdims=True)
        cwy = jnp.max(jnp.where(rowsel, gy, neg_inf), axis=1, keepdims=True)
        cwz = jnp.max(jnp.where(rowsel, gz, neg_inf), axis=1, keepdims=True)
        cwl = jnp.max(jnp.where(rowsel, gl, neg_inf), axis=1, keepdims=True)
        m = jnp.max(colmax, axis=2, keepdims=True)
        selc = colmax == m
        stack = jnp.concatenate([cwx, cwy, cwz, cwl], axis=1)
        win = jnp.max(jnp.where(selc, stack, neg_inf), axis=2,
                      keepdims=True)
        for b in range(_B):
            fps_ref[b, pl.ds(i, 1), 0:1] = win[b, 0:1, :]
            fps_ref[b, pl.ds(i, 1), 1:2] = win[b, 1:2, :]
            fps_ref[b, pl.ds(i, 1), 2:3] = win[b, 2:3, :]
            fps_ref[b, pl.ds(i, 1), 3:4] = win[b, 3:4, :]
        return (dists, win[:, 0:1, :], win[:, 1:2, :], win[:, 2:3, :])

    init = (jnp.full((_B, 32, 128), 1e10, f32),
            gtT_ref[:, 0:1, 0:1], gtT_ref[:, 1:2, 0:1], gtT_ref[:, 2:3, 0:1])
    jax.lax.fori_loop(1, _NFPS, fps_body, init, unroll=8)

    lane_gt = jax.lax.broadcasted_iota(jnp.int32, (_RB, _NGT), 1)
    lane8 = jax.lax.broadcasted_iota(jnp.int32, (_RB, _NCLS), 1)
    sum_d1_2 = f32(0)
    sum_d2_2 = f32(0)
    seg2_sum = f32(0)
    for b in range(_B):
        gxc = gtT_ref[b, 0:1, :]
        gyc = gtT_ref[b, 1:2, :]
        gzc = gtT_ref[b, 2:3, :]
        glc = gtT_ref[b, 3:4, :]
        b2 = gxc * gxc + gyc * gyc + gzc * gzc
        gxc16 = _r16(gxc)
        gyc16 = _r16(gyc)
        gzc16 = _r16(gzc)
        key_gt = lane_gt * _NCLS + glc.astype(jnp.int32)
        colmin = jnp.full((1, _NGT), jnp.inf, f32)
        for rb in range(_NP2 // _RB):
            r0 = rb * _RB
            xr = p2_ref[b, r0:r0 + _RB, 0:1]
            yr = p2_ref[b, r0:r0 + _RB, 1:2]
            zr = p2_ref[b, r0:r0 + _RB, 2:3]
            a2 = xr * xr + yr * yr + zr * zr
            ab = _r16(xr) * gxc16 + _r16(yr) * gyc16 + _r16(zr) * gzc16
            d = jnp.maximum(a2 + b2 - 2.0 * ab, 0.0)
            rmin = jnp.min(d, axis=1, keepdims=True)
            kmin = jnp.min(jnp.where(d == rmin, key_gt, _NGT * _NCLS),
                           axis=1, keepdims=True)
            lab = (kmin & (_NCLS - 1)).astype(f32)
            glab_ref[b, r0:r0 + _RB, :] = lab
            sum_d1_2 = sum_d1_2 + jnp.sum(jnp.sqrt(rmin))
            colmin = jnp.minimum(colmin, jnp.min(d, axis=0, keepdims=True))
            logits = p2_ref[b, r0:r0 + _RB, 3:3 + _NCLS]
            mx = jnp.max(logits, axis=1, keepdims=True)
            sh = logits - mx
            logp = sh - jnp.log(jnp.sum(jnp.exp(sh), axis=1, keepdims=True))
            labi = kmin & (_NCLS - 1)
            logpt = jnp.sum(jnp.where(lane8 == labi, logp, 0.0), axis=1,
                            keepdims=True)
            pt = jnp.exp(logpt)
            seg2_sum = seg2_sum + jnp.sum(-((1.0 - pt) ** gamma) * logpt)
        sum_d2_2 = sum_d2_2 + jnp.sum(jnp.sqrt(colmin))
    cd2 = (sum_d1_2 / (_B * _NP2) + sum_d2_2 / (_B * _NGT)) / 2.0
    seg2 = seg2_sum / (_B * _NP2)

    row_iota = jax.lax.broadcasted_iota(jnp.int32, (_RB, _NP1), 0)
    sub8 = jax.lax.broadcasted_iota(jnp.int32, (_NCLS, _NP1), 0)
    sum_d1_1 = f32(0)
    sum_d2_1 = f32(0)
    seg1_sum = f32(0)
    for b in range(_B):
        pxc = p1T_ref[b, 0:1, :]
        pyc = p1T_ref[b, 1:2, :]
        pzc = p1T_ref[b, 2:3, :]
        c2 = pxc * pxc + pyc * pyc + pzc * pzc
        pxc16 = _r16(pxc)
        pyc16 = _r16(pyc)
        pzc16 = _r16(pzc)
        colmin = jnp.full((1, _NP1), jnp.inf, f32)
        colkey = jnp.zeros((1, _NP1), jnp.int32)
        for rb in range(_NFPS // _RB):
            r0 = rb * _RB
            xr = fps_ref[b, r0:r0 + _RB, 0:1]
            yr = fps_ref[b, r0:r0 + _RB, 1:2]
            zr = fps_ref[b, r0:r0 + _RB, 2:3]
            lr = fps_ref[b, r0:r0 + _RB, 3:4]
            a2 = xr * xr + yr * yr + zr * zr
            ab = _r16(xr) * pxc16 + _r16(yr) * pyc16 + _r16(zr) * pzc16
            d = jnp.maximum(a2 + c2 - 2.0 * ab, 0.0)
            rmin = jnp.min(d, axis=1, keepdims=True)
            sum_d2_1 = sum_d2_1 + jnp.sum(jnp.sqrt(rmin))
            bmin = jnp.min(d, axis=0, keepdims=True)
            keys = row_iota * _NCLS + lr.astype(jnp.int32)
            bkey = jnp.min(jnp.where(d == bmin, keys, _NFPS * _NCLS),
                           axis=0, keepdims=True)
            upd = bmin < colmin
            colkey = jnp.where(upd, bkey, colkey)
            colmin = jnp.where(upd, bmin, colmin)
        sum_d1_1 = sum_d1_1 + jnp.sum(jnp.sqrt(colmin))
        logits = p1T_ref[b, 3:3 + _NCLS, :]
        mx = jnp.max(logits, axis=0, keepdims=True)
        sh = logits - mx
        logp = sh - jnp.log(jnp.sum(jnp.exp(sh), axis=0, keepdims=True))
        labi = colkey & (_NCLS - 1)
        logpt = jnp.sum(jnp.where(sub8 == labi, logp, 0.0), axis=0,
                        keepdims=True)
        pt = jnp.exp(logpt)
        seg1_sum = seg1_sum + jnp.sum(-((1.0 - pt) ** gamma) * logpt)
    cd1 = (sum_d1_1 / (_B * _NP1) + sum_d2_1 / (_B * _NFPS)) / 2.0
    seg1 = seg1_sum / (_B * _NP1)

    loss_ref[:, :] = ((cd1 + cd2) * 1000.0 + (seg1 + seg2) * 100.0).reshape(1, 1)
    cd2_ref[:, :] = cd2.reshape(1, 1)
    seg2_ref[:, :] = seg2.reshape(1, 1)


@functools.partial(jax.jit, static_argnames=("interpret",))
def _run(gamma, gt, gtT, gtR, p2, p1T, interpret=False):
    f32 = jnp.float32
    out_shapes = (
        jax.ShapeDtypeStruct((1, 1), f32),
        jax.ShapeDtypeStruct((1, 1), f32),
        jax.ShapeDtypeStruct((1, 1), f32),
        jax.ShapeDtypeStruct((_B, _NP2, 1), f32),
    )
    return pl.pallas_call(
        _loss_body,
        out_shape=out_shapes,
        in_specs=[
            pl.BlockSpec(memory_space=pltpu.SMEM),
            pl.BlockSpec(memory_space=pltpu.VMEM),
            pl.BlockSpec(memory_space=pltpu.VMEM),
            pl.BlockSpec(memory_space=pltpu.VMEM),
            pl.BlockSpec(memory_space=pltpu.VMEM),
            pl.BlockSpec(memory_space=pltpu.VMEM),
        ],
        scratch_shapes=[pltpu.VMEM((_B, _NFPS, 4), f32)],
        interpret=interpret,
    )(gamma, gt, gtT, gtR, p2, p1T)


def kernel(p0, p1, p2, gt, epoch, interpret=False):
    del p0
    gamma = jnp.clip(5.0 * (epoch / 200.0), 0.0, 20.0)
    gamma = jnp.asarray(gamma, jnp.float32).reshape(1, 1)
    gtT = jnp.transpose(gt, (0, 2, 1))
    gtR = gtT.reshape(_B, 4, 32, 128)
    p1T = jnp.transpose(p1, (0, 2, 1))
    loss_all, cd2, seg2, glab = _run(gamma, gt, gtT, gtR, p2, p1T,
                                     interpret=interpret)
    pred_label = p2[:, :, 3:]
    return (loss_all.reshape(()), cd2.reshape(()), seg2.reshape(()),
            pred_label, glab.reshape(_B, _NP2))


# speedup vs baseline: 3.3982x; 1.0002x over previous
import functools

import jax
import jax.numpy as jnp
from jax.experimental import pallas as pl
from jax.experimental.pallas import tpu as pltpu

_B = 2
_NGT = 4096
_NP1 = 2048
_NP2 = 4096
_NFPS = 2048
_NCLS = 8
_RB = 512


def _r16(x):
    return x.astype(jnp.bfloat16).astype(jnp.float32)


def _loss_body(gamma_ref, gt_ref, gtT_ref, gtR_ref, p2_ref, p1T_ref,
               loss_ref, cd2_ref, seg2_ref, glab_ref, fps_ref):
    f32 = jnp.float32
    gamma = gamma_ref[0, 0]

    neg_inf = f32(-jnp.inf)
    for b in range(_B):
        fps_ref[b, 0:1, :] = gt_ref[b, 0:1, :]

    def fps_body(i, carry):
        dists, xl, yl, zl = carry
        gx = gtR_ref[:, 0]
        gy = gtR_ref[:, 1]
        gz = gtR_ref[:, 2]
        gl = gtR_ref[:, 3]
        dx = gx - xl
        dy = gy - yl
        dz = gz - zl
        d = dx * dx + dy * dy + dz * dz
        dists = jnp.minimum(dists, d)
        colmax = jnp.max(dists, axis=1, keepdims=True)
        rowsel = dists == colmax
        cwx = jnp.max(jnp.where(rowsel, gx, neg_inf), axis=1, keepdims=True)
        cwy = jnp.max(jnp.where(rowsel, gy, neg_inf), axis=1, keepdims=True)
        cwz = jnp.max(jnp.where(rowsel, gz, neg_inf), axis=1, keepdims=True)
        cwl = jnp.max(jnp.where(rowsel, gl, neg_inf), axis=1, keepdims=True)
        m = jnp.max(colmax, axis=2, keepdims=True)
        selc = colmax == m
        stack = jnp.concatenate([cwx, cwy, cwz, cwl], axis=1)
        win = jnp.max(jnp.where(selc, stack, neg_inf), axis=2,
                      keepdims=True)
        for b in range(_B):
            fps_ref[b, pl.ds(i, 1), 0:1] = win[b, 0:1, :]
            fps_ref[b, pl.ds(i, 1), 1:2] = win[b, 1:2, :]
            fps_ref[b, pl.ds(i, 1), 2:3] = win[b, 2:3, :]
            fps_ref[b, pl.ds(i, 1), 3:4] = win[b, 3:4, :]
        return (dists, win[:, 0:1, :], win[:, 1:2, :], win[:, 2:3, :])

    init = (jnp.full((_B, 32, 128), 1e10, f32),
            gtT_ref[:, 0:1, 0:1], gtT_ref[:, 1:2, 0:1], gtT_ref[:, 2:3, 0:1])
    jax.lax.fori_loop(1, _NFPS, fps_body, init, unroll=8)

    lane_gt = jax.lax.broadcasted_iota(jnp.int32, (_RB, _NGT), 1)
    lane8 = jax.lax.broadcasted_iota(jnp.int32, (_RB, _NCLS), 1)
    sum_d1_2 = f32(0)
    sum_d2_2 = f32(0)
    seg2_sum = f32(0)
    for b in range(_B):
        gxc = gtT_ref[b, 0:1, :]
        gyc = gtT_ref[b, 1:2, :]
        gzc = gtT_ref[b, 2:3, :]
        glc = gtT_ref[b, 3:4, :]
        b2 = gxc * gxc + gyc * gyc + gzc * gzc
        gxc16 = _r16(gxc)
        gyc16 = _r16(gyc)
        gzc16 = _r16(gzc)
        key_gt = lane_gt * _NCLS + glc.astype(jnp.int32)
        colmin = jnp.full((1, _NGT), jnp.inf, f32)
        for rb in range(_NP2 // _RB):
            r0 = rb * _RB
            xr = p2_ref[b, r0:r0 + _RB, 0:1]
            yr = p2_ref[b, r0:r0 + _RB, 1:2]
            zr = p2_ref[b, r0:r0 + _RB, 2:3]
            a2 = xr * xr + yr * yr + zr * zr
            ab = _r16(xr) * gxc16 + _r16(yr) * gyc16 + _r16(zr) * gzc16
            d = jnp.maximum(a2 + b2 - 2.0 * ab, 0.0)
            rmin = jnp.min(d, axis=1, keepdims=True)
            kmin = jnp.min(jnp.where(d == rmin, key_gt, _NGT * _NCLS),
                           axis=1, keepdims=True)
            lab = (kmin & (_NCLS - 1)).astype(f32)
            glab_ref[b, r0:r0 + _RB, :] = lab
            sum_d1_2 = sum_d1_2 + jnp.sum(jnp.sqrt(rmin))
            colmin = jnp.minimum(colmin, jnp.min(d, axis=0, keepdims=True))
            logits = p2_ref[b, r0:r0 + _RB, 3:3 + _NCLS]
            mx = jnp.max(logits, axis=1, keepdims=True)
            sh = logits - mx
            logp = sh - jnp.log(jnp.sum(jnp.exp(sh), axis=1, keepdims=True))
            labi = kmin & (_NCLS - 1)
            logpt = jnp.sum(jnp.where(lane8 == labi, logp, 0.0), axis=1,
                            keepdims=True)
            pt = jnp.exp(logpt)
            seg2_sum = seg2_sum + jnp.sum(-((1.0 - pt) ** gamma) * logpt)
        sum_d2_2 = sum_d2_2 + jnp.sum(jnp.sqrt(colmin))
    cd2 = (sum_d1_2 / (_B * _NP2) + sum_d2_2 / (_B * _NGT)) / 2.0
    seg2 = seg2_sum / (_B * _NP2)

    row_iota = jax.lax.broadcasted_iota(jnp.int32, (_RB, _NP1), 0)
    sub8 = jax.lax.broadcasted_iota(jnp.int32, (_NCLS, _NP1), 0)
    sum_d1_1 = f32(0)
    sum_d2_1 = f32(0)
    seg1_sum = f32(0)
    for b in range(_B):
        pxc = p1T_ref[b, 0:1, :]
        pyc = p1T_ref[b, 1:2, :]
        pzc = p1T_ref[b, 2:3, :]
        c2 = pxc * pxc + pyc * pyc + pzc * pzc
        pxc16 = _r16(pxc)
        pyc16 = _r16(pyc)
        pzc16 = _r16(pzc)
        colmin = jnp.full((1, _NP1), jnp.inf, f32)
        colkey = jnp.zeros((1, _NP1), jnp.int32)
        for rb in range(_NFPS // _RB):
            r0 = rb * _RB
            xr = fps_ref[b, r0:r0 + _RB, 0:1]
            yr = fps_ref[b, r0:r0 + _RB, 1:2]
            zr = fps_ref[b, r0:r0 + _RB, 2:3]
            lr = fps_ref[b, r0:r0 + _RB, 3:4]
            a2 = xr * xr + yr * yr + zr * zr
            ab = _r16(xr) * pxc16 + _r16(yr) * pyc16 + _r16(zr) * pzc16
            d = jnp.maximum(a2 + c2 - 2.0 * ab, 0.0)
            rmin = jnp.min(d, axis=1, keepdims=True)
            sum_d2_1 = sum_d2_1 + jnp.sum(jnp.sqrt(rmin))
            bmin = jnp.min(d, axis=0, keepdims=True)
            keys = row_iota * _NCLS + lr.astype(jnp.int32)
            bkey = jnp.min(jnp.where(d == bmin, keys, _NFPS * _NCLS),
                           axis=0, keepdims=True)
            upd = bmin < colmin
            colkey = jnp.where(upd, bkey, colkey)
            colmin = jnp.where(upd, bmin, colmin)
        sum_d1_1 = sum_d1_1 + jnp.sum(jnp.sqrt(colmin))
        logits = p1T_ref[b, 3:3 + _NCLS, :]
        mx = jnp.max(logits, axis=0, keepdims=True)
        sh = logits - mx
        logp = sh - jnp.log(jnp.sum(jnp.exp(sh), axis=0, keepdims=True))
        labi = colkey & (_NCLS - 1)
        logpt = jnp.sum(jnp.where(sub8 == labi, logp, 0.0), axis=0,
                        keepdims=True)
        pt = jnp.exp(logpt)
        seg1_sum = seg1_sum + jnp.sum(-((1.0 - pt) ** gamma) * logpt)
    cd1 = (sum_d1_1 / (_B * _NP1) + sum_d2_1 / (_B * _NFPS)) / 2.0
    seg1 = seg1_sum / (_B * _NP1)

    loss_ref[:, :] = ((cd1 + cd2) * 1000.0 + (seg1 + seg2) * 100.0).reshape(1, 1)
    cd2_ref[:, :] = cd2.reshape(1, 1)
    seg2_ref[:, :] = seg2.reshape(1, 1)


@jax.jit
def _run(gamma, gt, gtT, gtR, p2, p1T):
    f32 = jnp.float32
    out_shapes = (
        jax.ShapeDtypeStruct((1, 1), f32),
        jax.ShapeDtypeStruct((1, 1), f32),
        jax.ShapeDtypeStruct((1, 1), f32),
        jax.ShapeDtypeStruct((_B, _NP2, 1), f32),
    )
    return pl.pallas_call(
        _loss_body,
        out_shape=out_shapes,
        in_specs=[
            pl.BlockSpec(memory_space=pltpu.SMEM),
            pl.BlockSpec(memory_space=pltpu.VMEM),
            pl.BlockSpec(memory_space=pltpu.VMEM),
            pl.BlockSpec(memory_space=pltpu.VMEM),
            pl.BlockSpec(memory_space=pltpu.VMEM),
            pl.BlockSpec(memory_space=pltpu.VMEM),
        ],
        scratch_shapes=[pltpu.VMEM((_B, _NFPS, 4), f32)],
    )(gamma, gt, gtT, gtR, p2, p1T)


def kernel(p0, p1, p2, gt, epoch):
    del p0
    gamma = jnp.clip(5.0 * (epoch / 200.0), 0.0, 20.0)
    gamma = jnp.asarray(gamma, jnp.float32).reshape(1, 1)
    gtT = jnp.transpose(gt, (0, 2, 1))
    gtR = gtT.reshape(_B, 4, 32, 128)
    p1T = jnp.transpose(p1, (0, 2, 1))
    loss_all, cd2, seg2, glab = _run(gamma, gt, gtT, gtR, p2, p1T)
    pred_label = p2[:, :, 3:]
    return (loss_all.reshape(()), cd2.reshape(()), seg2.reshape(()),
            pred_label, glab.reshape(_B, _NP2))
